# baseline (speedup 1.0000x reference)
import jax
import jax.numpy as jnp
from jax.experimental import pallas as pl
from jax.experimental.pallas import tpu as pltpu
from jax.experimental.pallas import tpu_sc as plsc

B, T, D = 1, 2048, 768
FF = 1536
E = 8
N = B * T
K = 2
NK = N * K

BLK = 128
NBLK = NK // BLK + E
PAD_N = NBLK * BLK
GW = 128
BT = 256


def _silu(v):
    return v * jax.nn.sigmoid(v)


def _dot(a, b):
    return jax.lax.dot_general(a, b, (((1,), (0,)), ((), ())),
                               preferred_element_type=jnp.float32)


def _sc_gather(table, indices):
    if table.dtype == jnp.bfloat16:
        w = table.shape[1]
        t32 = jax.lax.bitcast_convert_type(
            table.reshape(-1, w // 2, 2), jnp.int32)
        out32 = _sc_gather(t32, indices)
        return jax.lax.bitcast_convert_type(
            out32, jnp.bfloat16).reshape(-1, w)
    M = indices.shape[0]
    W = table.shape[1]
    idx2 = indices.reshape(1, M)
    mesh = plsc.VectorSubcoreMesh(core_axis_name="core",
                                  subcore_axis_name="subcore")

    @pl.kernel(out_type=jax.ShapeDtypeStruct((M, W), table.dtype),
               mesh=mesh)
    def kern(x_hbm, i_hbm, o_hbm):
        def body(i_vmem, o_vmem):
            pltpu.sync_copy(x_hbm.at[i_vmem.at[0]], o_vmem)

        pltpu.emit_pipeline(
            body,
            grid=(M // GW,),
            in_specs=[pl.BlockSpec((1, GW), lambda i: (0, i))],
            out_specs=[pl.BlockSpec((GW, W), lambda i: (i, 0))],
            core_axis_name=("core", "subcore"),
            dimension_semantics=(pltpu.PARALLEL,),
        )(i_hbm, o_hbm)

    return kern(table, idx2)


def _router_kernel(xb_ref, rw_ref, vals_ref, idx_ref):
    logits = _dot(xb_ref[...], rw_ref[...])
    m = jnp.max(logits, axis=-1, keepdims=True)
    p = jnp.exp(logits - m)
    p = p / jnp.sum(p, axis=-1, keepdims=True)
    iota = jax.lax.broadcasted_iota(jnp.int32, p.shape, 1)
    m1 = jnp.max(p, axis=-1, keepdims=True)
    i1 = jnp.min(jnp.where(p == m1, iota, E), axis=-1, keepdims=True)
    pm = jnp.where(iota == i1, -jnp.inf, p)
    m2 = jnp.max(pm, axis=-1, keepdims=True)
    i2 = jnp.min(jnp.where(pm == m2, iota, E), axis=-1, keepdims=True)
    vals_ref[...] = jnp.concatenate([m1, m2], axis=1)
    idx_ref[...] = jnp.concatenate([i1, i2], axis=1)


def _shared_kernel(xb_ref, gu_ref, dw_ref, sg_ref, sh_ref):
    xb = xb_ref[...]
    gu = _dot(xb, gu_ref[...])
    h = (_silu(gu[:, :FF]) * gu[:, FF:]).astype(jnp.bfloat16)
    sh = _dot(h, dw_ref[...])
    sgl = _dot(xb, sg_ref[...])
    sh_ref[...] = sh * jax.nn.sigmoid(sgl)


def _gemm_kernel(be_ref, xs_ref, wt_ref, w1_ref, w2_ref, ys_ref):
    del be_ref
    h = _dot(xs_ref[...], w1_ref[0])
    hb = _silu(h).astype(jnp.bfloat16)
    o = _dot(hb, w2_ref[0])
    ys_ref[...] = (o * wt_ref[...]).astype(jnp.bfloat16)


def _combine_kernel(g0_ref, g1_ref, sh_ref, out_ref):
    out_ref[...] = (g0_ref[...].astype(jnp.float32) +
                    g1_ref[...].astype(jnp.float32) + sh_ref[...])


def kernel(x, router_w, w1, w2, gate_up_w, down_w, shared_gate_w):
    Bv, Tv, Dv = x.shape
    flat = x.reshape(N, D)
    xb = flat.astype(jnp.bfloat16)

    vals, idx = pl.pallas_call(
        _router_kernel,
        grid=(N // 512,),
        in_specs=[
            pl.BlockSpec((512, D), lambda t: (t, 0)),
            pl.BlockSpec((D, E), lambda t: (0, 0)),
        ],
        out_specs=[
            pl.BlockSpec((512, 2), lambda t: (t, 0)),
            pl.BlockSpec((512, 2), lambda t: (t, 0)),
        ],
        out_shape=[
            jax.ShapeDtypeStruct((N, 2), jnp.float32),
            jax.ShapeDtypeStruct((N, 2), jnp.int32),
        ],
    )(xb, router_w.astype(jnp.bfloat16))

    idx_f = idx.reshape(-1)
    vals_f = vals.reshape(-1)
    oh = (idx_f[:, None] == jnp.arange(E, dtype=jnp.int32)[None, :]
          ).astype(jnp.int32)
    counts = jnp.sum(oh, axis=0)
    rank = jnp.sum((jnp.cumsum(oh, axis=0) - oh) * oh, axis=1)
    pad_counts = ((counts + BLK - 1) // BLK) * BLK
    pad_start = jnp.concatenate(
        [jnp.zeros((1,), jnp.int32),
         jnp.cumsum(pad_counts)[:-1].astype(jnp.int32)])
    p = pad_start[idx_f] + rank
    tok = jnp.arange(NK, dtype=jnp.int32) // K
    tok_slot = jnp.zeros((PAD_N,), jnp.int32).at[p].set(tok)
    wt_slot = jnp.zeros((PAD_N,), jnp.float32).at[p].set(vals_f)
    blk_start = pad_start // BLK
    blk_expert = (jnp.sum((jnp.arange(NBLK, dtype=jnp.int32)[:, None] >=
                           blk_start[None, :]).astype(jnp.int32), axis=1)
                  - 1).astype(jnp.int32)

    xs = _sc_gather(xb, tok_slot)

    ys = pl.pallas_call(
        _gemm_kernel,
        grid_spec=pltpu.PrefetchScalarGridSpec(
            num_scalar_prefetch=1,
            grid=(NBLK,),
            in_specs=[
                pl.BlockSpec((BLK, D), lambda b, be: (b, 0)),
                pl.BlockSpec((BLK, 1), lambda b, be: (b, 0)),
                pl.BlockSpec((1, D, FF), lambda b, be: (be[b], 0, 0)),
                pl.BlockSpec((1, FF, D), lambda b, be: (be[b], 0, 0)),
            ],
            out_specs=pl.BlockSpec((BLK, D), lambda b, be: (b, 0)),
        ),
        out_shape=jax.ShapeDtypeStruct((PAD_N, D), jnp.bfloat16),
        compiler_params=pltpu.CompilerParams(
            dimension_semantics=("arbitrary",)),
    )(blk_expert, xs, wt_slot.reshape(PAD_N, 1),
      w1.astype(jnp.bfloat16), w2.astype(jnp.bfloat16))

    sh = pl.pallas_call(
        _shared_kernel,
        grid=(N // BT,),
        in_specs=[
            pl.BlockSpec((BT, D), lambda t: (t, 0)),
            pl.BlockSpec((D, 2 * FF), lambda t: (0, 0)),
            pl.BlockSpec((FF, D), lambda t: (0, 0)),
            pl.BlockSpec((D, 1), lambda t: (0, 0)),
        ],
        out_specs=pl.BlockSpec((BT, D), lambda t: (t, 0)),
        out_shape=jax.ShapeDtypeStruct((N, D), jnp.float32),
    )(xb, gate_up_w.astype(jnp.bfloat16), down_w.astype(jnp.bfloat16),
      shared_gate_w.astype(jnp.bfloat16))

    p2 = p.reshape(N, K)
    p_cat = jnp.concatenate([p2[:, 0], p2[:, 1]])
    g = _sc_gather(ys, p_cat)

    nb = N // BT
    out = pl.pallas_call(
        _combine_kernel,
        grid=(nb,),
        in_specs=[
            pl.BlockSpec((BT, D), lambda t: (t, 0)),
            pl.BlockSpec((BT, D), lambda t: (t + nb, 0)),
            pl.BlockSpec((BT, D), lambda t: (t, 0)),
        ],
        out_specs=pl.BlockSpec((BT, D), lambda t: (t, 0)),
        out_shape=jax.ShapeDtypeStruct((N, D), jnp.float32),
    )(g, g, sh)

    return out.reshape(Bv, Tv, Dv)


# speedup vs baseline: 1.6600x; 1.6600x over previous
import jax
import jax.numpy as jnp
from jax.experimental import pallas as pl
from jax.experimental.pallas import tpu as pltpu
from jax.experimental.pallas import tpu_sc as plsc

B, T, D = 1, 2048, 768
FF = 1536
E = 8
N = B * T
K = 2
NK = N * K

BLK = 128
NBLK = NK // BLK + E
PAD_N = NBLK * BLK
GW = 128
BT = 256


def _silu(v):
    return v * jax.nn.sigmoid(v)


def _dot(a, b):
    return jax.lax.dot_general(a, b, (((1,), (0,)), ((), ())),
                               preferred_element_type=jnp.float32)


def _sc_gather(table, indices):
    SPLIT = 2
    R, W0 = table.shape
    table = table.reshape(R * SPLIT, W0 // SPLIT)
    indices = (indices[:, None] * SPLIT +
               jnp.arange(SPLIT, dtype=jnp.int32)[None, :]).reshape(-1)
    M = indices.shape[0]
    W = table.shape[1]
    idx2 = indices.reshape(1, M)
    mesh = plsc.VectorSubcoreMesh(core_axis_name="core",
                                  subcore_axis_name="subcore")

    @pl.kernel(out_type=jax.ShapeDtypeStruct((M, W), table.dtype),
               mesh=mesh)
    def kern(x_hbm, i_hbm, o_hbm):
        def body(i_vmem, o_vmem):
            pltpu.sync_copy(x_hbm.at[i_vmem.at[0]], o_vmem)

        pltpu.emit_pipeline(
            body,
            grid=(M // GW,),
            in_specs=[pl.BlockSpec((1, GW), lambda i: (0, i))],
            out_specs=[pl.BlockSpec((GW, W), lambda i: (i, 0))],
            core_axis_name=("core", "subcore"),
            dimension_semantics=(pltpu.PARALLEL,),
        )(i_hbm, o_hbm)

    return kern(table, idx2).reshape(M // SPLIT, W0)


def _router_kernel(xb_ref, rw_ref, vals_ref, idx_ref):
    logits = _dot(xb_ref[...], rw_ref[...])
    m = jnp.max(logits, axis=-1, keepdims=True)
    p = jnp.exp(logits - m)
    p = p / jnp.sum(p, axis=-1, keepdims=True)
    iota = jax.lax.broadcasted_iota(jnp.int32, p.shape, 1)
    m1 = jnp.max(p, axis=-1, keepdims=True)
    i1 = jnp.min(jnp.where(p == m1, iota, E), axis=-1, keepdims=True)
    pm = jnp.where(iota == i1, -jnp.inf, p)
    m2 = jnp.max(pm, axis=-1, keepdims=True)
    i2 = jnp.min(jnp.where(pm == m2, iota, E), axis=-1, keepdims=True)
    vals_ref[...] = jnp.concatenate([m1, m2], axis=1)
    idx_ref[...] = jnp.concatenate([i1, i2], axis=1)


def _shared_kernel(xb_ref, gu_ref, dw_ref, sg_ref, sh_ref):
    xb = xb_ref[...]
    gu = _dot(xb, gu_ref[...])
    h = (_silu(gu[:, :FF]) * gu[:, FF:]).astype(jnp.bfloat16)
    sh = _dot(h, dw_ref[...])
    sgl = _dot(xb, sg_ref[...])
    sh_ref[...] = sh * jax.nn.sigmoid(sgl)


def _gemm_kernel(be_ref, xs_ref, wt_ref, w1_ref, w2_ref, ys_ref):
    del be_ref
    h = _dot(xs_ref[...].astype(jnp.bfloat16), w1_ref[0])
    hb = _silu(h).astype(jnp.bfloat16)
    o = _dot(hb, w2_ref[0])
    ys_ref[...] = o * wt_ref[...]


def _combine_kernel(g0_ref, g1_ref, sh_ref, out_ref):
    out_ref[...] = g0_ref[...] + g1_ref[...] + sh_ref[...]


def kernel(x, router_w, w1, w2, gate_up_w, down_w, shared_gate_w):
    Bv, Tv, Dv = x.shape
    flat = x.reshape(N, D)
    xb = flat.astype(jnp.bfloat16)

    vals, idx = pl.pallas_call(
        _router_kernel,
        grid=(N // 512,),
        in_specs=[
            pl.BlockSpec((512, D), lambda t: (t, 0)),
            pl.BlockSpec((D, E), lambda t: (0, 0)),
        ],
        out_specs=[
            pl.BlockSpec((512, 2), lambda t: (t, 0)),
            pl.BlockSpec((512, 2), lambda t: (t, 0)),
        ],
        out_shape=[
            jax.ShapeDtypeStruct((N, 2), jnp.float32),
            jax.ShapeDtypeStruct((N, 2), jnp.int32),
        ],
    )(xb, router_w.astype(jnp.bfloat16))

    idx_f = idx.reshape(-1)
    vals_f = vals.reshape(-1)
    oh = (idx_f[:, None] == jnp.arange(E, dtype=jnp.int32)[None, :]
          ).astype(jnp.int32)
    counts = jnp.sum(oh, axis=0)
    rank = jnp.sum((jnp.cumsum(oh, axis=0) - oh) * oh, axis=1)
    pad_counts = ((counts + BLK - 1) // BLK) * BLK
    pad_start = jnp.concatenate(
        [jnp.zeros((1,), jnp.int32),
         jnp.cumsum(pad_counts)[:-1].astype(jnp.int32)])
    p = pad_start[idx_f] + rank
    tok = jnp.arange(NK, dtype=jnp.int32) // K
    tok_slot = jnp.zeros((PAD_N,), jnp.int32).at[p].set(tok)
    wt_slot = jnp.zeros((PAD_N,), jnp.float32).at[p].set(vals_f)
    blk_start = pad_start // BLK
    blk_expert = (jnp.sum((jnp.arange(NBLK, dtype=jnp.int32)[:, None] >=
                           blk_start[None, :]).astype(jnp.int32), axis=1)
                  - 1).astype(jnp.int32)

    xs = _sc_gather(flat, tok_slot)

    ys = pl.pallas_call(
        _gemm_kernel,
        grid_spec=pltpu.PrefetchScalarGridSpec(
            num_scalar_prefetch=1,
            grid=(NBLK,),
            in_specs=[
                pl.BlockSpec((BLK, D), lambda b, be: (b, 0)),
                pl.BlockSpec((BLK, 1), lambda b, be: (b, 0)),
                pl.BlockSpec((1, D, FF), lambda b, be: (be[b], 0, 0)),
                pl.BlockSpec((1, FF, D), lambda b, be: (be[b], 0, 0)),
            ],
            out_specs=pl.BlockSpec((BLK, D), lambda b, be: (b, 0)),
        ),
        out_shape=jax.ShapeDtypeStruct((PAD_N, D), jnp.float32),
        compiler_params=pltpu.CompilerParams(
            dimension_semantics=("arbitrary",)),
    )(blk_expert, xs, wt_slot.reshape(PAD_N, 1),
      w1.astype(jnp.bfloat16), w2.astype(jnp.bfloat16))

    sh = pl.pallas_call(
        _shared_kernel,
        grid=(N // BT,),
        in_specs=[
            pl.BlockSpec((BT, D), lambda t: (t, 0)),
            pl.BlockSpec((D, 2 * FF), lambda t: (0, 0)),
            pl.BlockSpec((FF, D), lambda t: (0, 0)),
            pl.BlockSpec((D, 1), lambda t: (0, 0)),
        ],
        out_specs=pl.BlockSpec((BT, D), lambda t: (t, 0)),
        out_shape=jax.ShapeDtypeStruct((N, D), jnp.float32),
    )(xb, gate_up_w.astype(jnp.bfloat16), down_w.astype(jnp.bfloat16),
      shared_gate_w.astype(jnp.bfloat16))

    p2 = p.reshape(N, K)
    p_cat = jnp.concatenate([p2[:, 0], p2[:, 1]])
    g = _sc_gather(ys, p_cat)

    nb = N // BT
    out = pl.pallas_call(
        _combine_kernel,
        grid=(nb,),
        in_specs=[
            pl.BlockSpec((BT, D), lambda t: (t, 0)),
            pl.BlockSpec((BT, D), lambda t: (t + nb, 0)),
            pl.BlockSpec((BT, D), lambda t: (t, 0)),
        ],
        out_specs=pl.BlockSpec((BT, D), lambda t: (t, 0)),
        out_shape=jax.ShapeDtypeStruct((N, D), jnp.float32),
    )(g, g, sh)

    return out.reshape(Bv, Tv, Dv)


# speedup vs baseline: 1.6897x; 1.0179x over previous
import jax
import jax.numpy as jnp
from jax.experimental import pallas as pl
from jax.experimental.pallas import tpu as pltpu
from jax.experimental.pallas import tpu_sc as plsc

B, T, D = 1, 2048, 768
FF = 1536
E = 8
N = B * T
K = 2
NK = N * K

BLK = 128
NBLK = NK // BLK + E
PAD_N = NBLK * BLK
GW = 128
BT = 256


def _silu(v):
    return v * jax.nn.sigmoid(v)


def _dot(a, b):
    return jax.lax.dot_general(a, b, (((1,), (0,)), ((), ())),
                               preferred_element_type=jnp.float32)


def _sc_gather(table, indices):
    SPLIT = 2
    R, W0 = table.shape
    table = table.reshape(R * SPLIT, W0 // SPLIT)
    indices = (indices[:, None] * SPLIT +
               jnp.arange(SPLIT, dtype=jnp.int32)[None, :]).reshape(-1)
    M = indices.shape[0]
    W = table.shape[1]
    idx2 = indices.reshape(1, M)
    mesh = plsc.VectorSubcoreMesh(core_axis_name="core",
                                  subcore_axis_name="subcore")

    @pl.kernel(out_type=jax.ShapeDtypeStruct((M, W), table.dtype),
               mesh=mesh)
    def kern(x_hbm, i_hbm, o_hbm):
        def body(i_vmem, o_vmem):
            pltpu.sync_copy(x_hbm.at[i_vmem.at[0]], o_vmem)

        pltpu.emit_pipeline(
            body,
            grid=(M // GW,),
            in_specs=[pl.BlockSpec((1, GW), lambda i: (0, i))],
            out_specs=[pl.BlockSpec((GW, W), lambda i: (i, 0))],
            core_axis_name=("core", "subcore"),
            dimension_semantics=(pltpu.PARALLEL,),
        )(i_hbm, o_hbm)

    return kern(table, idx2).reshape(M // SPLIT, W0)


def _router_kernel(xb_ref, rw_ref, vals_ref, idx_ref):
    logits = _dot(xb_ref[...], rw_ref[...])
    m = jnp.max(logits, axis=-1, keepdims=True)
    p = jnp.exp(logits - m)
    p = p / jnp.sum(p, axis=-1, keepdims=True)
    iota = jax.lax.broadcasted_iota(jnp.int32, p.shape, 1)
    m1 = jnp.max(p, axis=-1, keepdims=True)
    i1 = jnp.min(jnp.where(p == m1, iota, E), axis=-1, keepdims=True)
    pm = jnp.where(iota == i1, -jnp.inf, p)
    m2 = jnp.max(pm, axis=-1, keepdims=True)
    i2 = jnp.min(jnp.where(pm == m2, iota, E), axis=-1, keepdims=True)
    vals_ref[...] = jnp.concatenate([m1, m2], axis=1)
    idx_ref[...] = jnp.concatenate([i1, i2], axis=1)


def _shared_kernel(xb_ref, gu_ref, dw_ref, sg_ref, sh_ref):
    xb = xb_ref[...]
    gu = _dot(xb, gu_ref[...])
    h = (_silu(gu[:, :FF]) * gu[:, FF:]).astype(jnp.bfloat16)
    sh = _dot(h, dw_ref[...])
    sgl = _dot(xb, sg_ref[...])
    sh_ref[...] = sh * jax.nn.sigmoid(sgl)


def _gemm_kernel(be_ref, xs_ref, po0_ref, po1_ref, va0_ref, va1_ref,
                 sh_ref, w1_ref, w2_ref, out_ref):
    del be_ref
    b = pl.program_id(0)
    h = _dot(xs_ref[...].astype(jnp.bfloat16), w1_ref[0])
    hb = _silu(h).astype(jnp.bfloat16)
    o = _dot(hb, w2_ref[0]).astype(jnp.bfloat16)
    si = b * BLK + jax.lax.broadcasted_iota(jnp.int32, (BLK, N), 0)
    q = (jnp.where(po0_ref[...] == si, va0_ref[...], 0.0) +
         jnp.where(po1_ref[...] == si, va1_ref[...], 0.0)
         ).astype(jnp.bfloat16)
    contrib = jax.lax.dot_general(q, o, (((0,), (0,)), ((), ())),
                                  preferred_element_type=jnp.float32)

    @pl.when(b == 0)
    def _():
        out_ref[...] = sh_ref[...]

    out_ref[...] += contrib


def kernel(x, router_w, w1, w2, gate_up_w, down_w, shared_gate_w):
    Bv, Tv, Dv = x.shape
    flat = x.reshape(N, D)
    xb = flat.astype(jnp.bfloat16)

    vals, idx = pl.pallas_call(
        _router_kernel,
        grid=(N // 512,),
        in_specs=[
            pl.BlockSpec((512, D), lambda t: (t, 0)),
            pl.BlockSpec((D, E), lambda t: (0, 0)),
        ],
        out_specs=[
            pl.BlockSpec((512, 2), lambda t: (t, 0)),
            pl.BlockSpec((512, 2), lambda t: (t, 0)),
        ],
        out_shape=[
            jax.ShapeDtypeStruct((N, 2), jnp.float32),
            jax.ShapeDtypeStruct((N, 2), jnp.int32),
        ],
    )(xb, router_w.astype(jnp.bfloat16))

    idx_f = idx.reshape(-1)
    vals_f = vals.reshape(-1)
    oh = (idx_f[:, None] == jnp.arange(E, dtype=jnp.int32)[None, :]
          ).astype(jnp.int32)
    counts = jnp.sum(oh, axis=0)
    rank = jnp.sum((jnp.cumsum(oh, axis=0) - oh) * oh, axis=1)
    pad_counts = ((counts + BLK - 1) // BLK) * BLK
    pad_start = jnp.concatenate(
        [jnp.zeros((1,), jnp.int32),
         jnp.cumsum(pad_counts)[:-1].astype(jnp.int32)])
    p = pad_start[idx_f] + rank
    tok = jnp.arange(NK, dtype=jnp.int32) // K
    tok_slot = jnp.zeros((PAD_N,), jnp.int32).at[p].set(tok)
    blk_start = pad_start // BLK
    blk_expert = (jnp.sum((jnp.arange(NBLK, dtype=jnp.int32)[:, None] >=
                           blk_start[None, :]).astype(jnp.int32), axis=1)
                  - 1).astype(jnp.int32)

    xs = _sc_gather(flat, tok_slot)

    sh = pl.pallas_call(
        _shared_kernel,
        grid=(N // BT,),
        in_specs=[
            pl.BlockSpec((BT, D), lambda t: (t, 0)),
            pl.BlockSpec((D, 2 * FF), lambda t: (0, 0)),
            pl.BlockSpec((FF, D), lambda t: (0, 0)),
            pl.BlockSpec((D, 1), lambda t: (0, 0)),
        ],
        out_specs=pl.BlockSpec((BT, D), lambda t: (t, 0)),
        out_shape=jax.ShapeDtypeStruct((N, D), jnp.float32),
    )(xb, gate_up_w.astype(jnp.bfloat16), down_w.astype(jnp.bfloat16),
      shared_gate_w.astype(jnp.bfloat16))

    p2 = p.reshape(N, K)
    v2 = vals_f.reshape(N, K)
    out = pl.pallas_call(
        _gemm_kernel,
        grid_spec=pltpu.PrefetchScalarGridSpec(
            num_scalar_prefetch=1,
            grid=(NBLK,),
            in_specs=[
                pl.BlockSpec((BLK, D), lambda b, be: (b, 0)),
                pl.BlockSpec((1, N), lambda b, be: (0, 0)),
                pl.BlockSpec((1, N), lambda b, be: (0, 0)),
                pl.BlockSpec((1, N), lambda b, be: (0, 0)),
                pl.BlockSpec((1, N), lambda b, be: (0, 0)),
                pl.BlockSpec((N, D), lambda b, be: (0, 0)),
                pl.BlockSpec((1, D, FF), lambda b, be: (be[b], 0, 0)),
                pl.BlockSpec((1, FF, D), lambda b, be: (be[b], 0, 0)),
            ],
            out_specs=pl.BlockSpec((N, D), lambda b, be: (0, 0)),
        ),
        out_shape=jax.ShapeDtypeStruct((N, D), jnp.float32),
        compiler_params=pltpu.CompilerParams(
            dimension_semantics=("arbitrary",)),
    )(blk_expert, xs,
      p2[:, 0].reshape(1, N), p2[:, 1].reshape(1, N),
      v2[:, 0].reshape(1, N), v2[:, 1].reshape(1, N),
      sh, w1.astype(jnp.bfloat16), w2.astype(jnp.bfloat16))

    return out.reshape(Bv, Tv, Dv)


# speedup vs baseline: 2.3251x; 1.3760x over previous
import jax
import jax.numpy as jnp
from jax.experimental import pallas as pl
from jax.experimental.pallas import tpu as pltpu
from jax.experimental.pallas import tpu_sc as plsc

B, T, D = 1, 2048, 768
FF = 1536
E = 8
N = B * T
K = 2
NK = N * K

BLK = 128
NBLK = NK // BLK + E
PAD_N = NBLK * BLK
GW = 128
BT = 256


def _silu(v):
    return v * jax.nn.sigmoid(v)


def _dot(a, b):
    return jax.lax.dot_general(a, b, (((1,), (0,)), ((), ())),
                               preferred_element_type=jnp.float32)


def _sc_gather(table, indices):
    SPLIT = 2
    R, W0 = table.shape
    table = table.reshape(R * SPLIT, W0 // SPLIT)
    indices = (indices[:, None] * SPLIT +
               jnp.arange(SPLIT, dtype=jnp.int32)[None, :]).reshape(-1)
    M = indices.shape[0]
    W = table.shape[1]
    idx2 = indices.reshape(1, M)
    mesh = plsc.VectorSubcoreMesh(core_axis_name="core",
                                  subcore_axis_name="subcore")

    @pl.kernel(out_type=jax.ShapeDtypeStruct((M, W), table.dtype),
               mesh=mesh)
    def kern(x_hbm, i_hbm, o_hbm):
        def body(i_vmem, o_vmem):
            pltpu.sync_copy(x_hbm.at[i_vmem.at[0]], o_vmem)

        pltpu.emit_pipeline(
            body,
            grid=(M // GW,),
            in_specs=[pl.BlockSpec((1, GW), lambda i: (0, i))],
            out_specs=[pl.BlockSpec((GW, W), lambda i: (i, 0))],
            core_axis_name=("core", "subcore"),
            dimension_semantics=(pltpu.PARALLEL,),
        )(i_hbm, o_hbm)

    return kern(table, idx2).reshape(M // SPLIT, W0)


def _router_kernel(xb_ref, rw_ref, vals_ref, idx_ref):
    logits = _dot(xb_ref[...], rw_ref[...])
    m = jnp.max(logits, axis=-1, keepdims=True)
    p = jnp.exp(logits - m)
    p = p / jnp.sum(p, axis=-1, keepdims=True)
    iota = jax.lax.broadcasted_iota(jnp.int32, p.shape, 1)
    m1 = jnp.max(p, axis=-1, keepdims=True)
    i1 = jnp.min(jnp.where(p == m1, iota, E), axis=-1, keepdims=True)
    pm = jnp.where(iota == i1, -jnp.inf, p)
    m2 = jnp.max(pm, axis=-1, keepdims=True)
    i2 = jnp.min(jnp.where(pm == m2, iota, E), axis=-1, keepdims=True)
    vals_ref[...] = jnp.concatenate([m1, m2], axis=1)
    idx_ref[...] = jnp.concatenate([i1, i2], axis=1)


def _shared_kernel(xb_ref, gu_ref, dw_ref, sg_ref, sh_ref):
    xb = xb_ref[...]
    gu = _dot(xb, gu_ref[...])
    h = (_silu(gu[:, :FF]) * gu[:, FF:]).astype(jnp.bfloat16)
    sh = _dot(h, dw_ref[...])
    sgl = _dot(xb, sg_ref[...])
    sh_ref[...] = sh * jax.nn.sigmoid(sgl)


def _gemm_kernel(be_ref, xb_ref, po0_ref, po1_ref, va0_ref, va1_ref,
                 sh_ref, w1_ref, w2_ref, out_ref):
    del be_ref
    b = pl.program_id(0)
    si = b * BLK + jax.lax.broadcasted_iota(jnp.int32, (BLK, N), 0)
    eq0 = po0_ref[...] == si
    eq1 = po1_ref[...] == si
    pm = (eq0 | eq1).astype(jnp.bfloat16)
    xs = _dot(pm, xb_ref[...]).astype(jnp.bfloat16)
    h = _dot(xs, w1_ref[0])
    hb = _silu(h).astype(jnp.bfloat16)
    o = _dot(hb, w2_ref[0]).astype(jnp.bfloat16)
    q = (jnp.where(eq0, va0_ref[...], 0.0) +
         jnp.where(eq1, va1_ref[...], 0.0)).astype(jnp.bfloat16)
    contrib = jax.lax.dot_general(q, o, (((0,), (0,)), ((), ())),
                                  preferred_element_type=jnp.float32)

    @pl.when(b == 0)
    def _():
        out_ref[...] = sh_ref[...]

    out_ref[...] += contrib


def kernel(x, router_w, w1, w2, gate_up_w, down_w, shared_gate_w):
    Bv, Tv, Dv = x.shape
    flat = x.reshape(N, D)
    xb = flat.astype(jnp.bfloat16)

    vals, idx = pl.pallas_call(
        _router_kernel,
        grid=(N // 512,),
        in_specs=[
            pl.BlockSpec((512, D), lambda t: (t, 0)),
            pl.BlockSpec((D, E), lambda t: (0, 0)),
        ],
        out_specs=[
            pl.BlockSpec((512, 2), lambda t: (t, 0)),
            pl.BlockSpec((512, 2), lambda t: (t, 0)),
        ],
        out_shape=[
            jax.ShapeDtypeStruct((N, 2), jnp.float32),
            jax.ShapeDtypeStruct((N, 2), jnp.int32),
        ],
    )(xb, router_w.astype(jnp.bfloat16))

    idx_f = idx.reshape(-1)
    vals_f = vals.reshape(-1)
    oh = (idx_f[:, None] == jnp.arange(E, dtype=jnp.int32)[None, :]
          ).astype(jnp.int32)
    counts = jnp.sum(oh, axis=0)
    rank = jnp.sum((jnp.cumsum(oh, axis=0) - oh) * oh, axis=1)
    pad_counts = ((counts + BLK - 1) // BLK) * BLK
    pad_start = jnp.concatenate(
        [jnp.zeros((1,), jnp.int32),
         jnp.cumsum(pad_counts)[:-1].astype(jnp.int32)])
    p = pad_start[idx_f] + rank
    blk_start = pad_start // BLK
    blk_expert = (jnp.sum((jnp.arange(NBLK, dtype=jnp.int32)[:, None] >=
                           blk_start[None, :]).astype(jnp.int32), axis=1)
                  - 1).astype(jnp.int32)

    sh = pl.pallas_call(
        _shared_kernel,
        grid=(N // BT,),
        in_specs=[
            pl.BlockSpec((BT, D), lambda t: (t, 0)),
            pl.BlockSpec((D, 2 * FF), lambda t: (0, 0)),
            pl.BlockSpec((FF, D), lambda t: (0, 0)),
            pl.BlockSpec((D, 1), lambda t: (0, 0)),
        ],
        out_specs=pl.BlockSpec((BT, D), lambda t: (t, 0)),
        out_shape=jax.ShapeDtypeStruct((N, D), jnp.float32),
    )(xb, gate_up_w.astype(jnp.bfloat16), down_w.astype(jnp.bfloat16),
      shared_gate_w.astype(jnp.bfloat16))

    p2 = p.reshape(N, K)
    v2 = vals_f.reshape(N, K)
    out = pl.pallas_call(
        _gemm_kernel,
        grid_spec=pltpu.PrefetchScalarGridSpec(
            num_scalar_prefetch=1,
            grid=(NBLK,),
            in_specs=[
                pl.BlockSpec((N, D), lambda b, be: (0, 0)),
                pl.BlockSpec((1, N), lambda b, be: (0, 0)),
                pl.BlockSpec((1, N), lambda b, be: (0, 0)),
                pl.BlockSpec((1, N), lambda b, be: (0, 0)),
                pl.BlockSpec((1, N), lambda b, be: (0, 0)),
                pl.BlockSpec((N, D), lambda b, be: (0, 0)),
                pl.BlockSpec((1, D, FF), lambda b, be: (be[b], 0, 0)),
                pl.BlockSpec((1, FF, D), lambda b, be: (be[b], 0, 0)),
            ],
            out_specs=pl.BlockSpec((N, D), lambda b, be: (0, 0)),
        ),
        out_shape=jax.ShapeDtypeStruct((N, D), jnp.float32),
        compiler_params=pltpu.CompilerParams(
            dimension_semantics=("arbitrary",)),
    )(blk_expert, xb,
      p2[:, 0].reshape(1, N), p2[:, 1].reshape(1, N),
      v2[:, 0].reshape(1, N), v2[:, 1].reshape(1, N),
      sh, w1.astype(jnp.bfloat16), w2.astype(jnp.bfloat16))

    return out.reshape(Bv, Tv, Dv)


# speedup vs baseline: 2.4049x; 1.0343x over previous
import jax
import jax.numpy as jnp
from jax.experimental import pallas as pl
from jax.experimental.pallas import tpu as pltpu
from jax.experimental.pallas import tpu_sc as plsc

B, T, D = 1, 2048, 768
FF = 1536
E = 8
N = B * T
K = 2
NK = N * K

BLK = 128
NBLK = NK // BLK + E
PAD_N = NBLK * BLK
GW = 128
BT = 256


def _silu(v):
    return v * jax.nn.sigmoid(v)


def _dot(a, b):
    return jax.lax.dot_general(a, b, (((1,), (0,)), ((), ())),
                               preferred_element_type=jnp.float32)


def _sc_gather(table, indices):
    SPLIT = 2
    R, W0 = table.shape
    table = table.reshape(R * SPLIT, W0 // SPLIT)
    indices = (indices[:, None] * SPLIT +
               jnp.arange(SPLIT, dtype=jnp.int32)[None, :]).reshape(-1)
    M = indices.shape[0]
    W = table.shape[1]
    idx2 = indices.reshape(1, M)
    mesh = plsc.VectorSubcoreMesh(core_axis_name="core",
                                  subcore_axis_name="subcore")

    @pl.kernel(out_type=jax.ShapeDtypeStruct((M, W), table.dtype),
               mesh=mesh)
    def kern(x_hbm, i_hbm, o_hbm):
        def body(i_vmem, o_vmem):
            pltpu.sync_copy(x_hbm.at[i_vmem.at[0]], o_vmem)

        pltpu.emit_pipeline(
            body,
            grid=(M // GW,),
            in_specs=[pl.BlockSpec((1, GW), lambda i: (0, i))],
            out_specs=[pl.BlockSpec((GW, W), lambda i: (i, 0))],
            core_axis_name=("core", "subcore"),
            dimension_semantics=(pltpu.PARALLEL,),
        )(i_hbm, o_hbm)

    return kern(table, idx2).reshape(M // SPLIT, W0)


def _router_kernel(xb_ref, rw_ref, po0_ref, po1_ref, va0_ref, va1_ref,
                   be_ref):
    logits = _dot(xb_ref[...], rw_ref[...])
    m = jnp.max(logits, axis=-1, keepdims=True)
    p = jnp.exp(logits - m)
    p = p / jnp.sum(p, axis=-1, keepdims=True)
    iota = jax.lax.broadcasted_iota(jnp.int32, p.shape, 1)
    m1 = jnp.max(p, axis=-1, keepdims=True)
    i1 = jnp.min(jnp.where(p == m1, iota, E), axis=-1, keepdims=True)
    pm = jnp.where(iota == i1, -jnp.inf, p)
    m2 = jnp.max(pm, axis=-1, keepdims=True)
    i2 = jnp.min(jnp.where(pm == m2, iota, E), axis=-1, keepdims=True)
    va0_ref[...] = m1
    va1_ref[...] = m2
    oh1 = iota == i1
    oh2 = iota == i2
    ohb = (oh1 | oh2).astype(jnp.int32)
    incl = ohb
    d = 1
    while d < N:
        shifted = jnp.concatenate(
            [jnp.zeros((d, E), jnp.int32), incl[:N - d, :]], axis=0)
        incl = incl + shifted
        d *= 2
    excl = incl - ohb
    counts = incl[N - 1:N, :]
    pad_counts = ((counts + BLK - 1) // BLK) * BLK
    pad_end = pad_counts
    d = 1
    while d < E:
        pad_end = pad_end + jnp.concatenate(
            [jnp.zeros((1, d), jnp.int32), pad_end[:, :E - d]], axis=1)
        d *= 2
    pad_start = pad_end - pad_counts
    slot = excl + pad_start
    po0_ref[...] = jnp.sum(jnp.where(oh1, slot, 0), axis=1, keepdims=True)
    po1_ref[...] = jnp.sum(jnp.where(oh2, slot, 0), axis=1, keepdims=True)
    bb = BLK * jax.lax.broadcasted_iota(jnp.int32, (NBLK, E), 0)
    be = jnp.sum((jnp.broadcast_to(pad_end, (NBLK, E)) <= bb
                  ).astype(jnp.int32), axis=1, keepdims=True)
    be_ref[...] = jnp.minimum(be, E - 1)


def _shared_kernel(xb_ref, gu_ref, dw_ref, sg_ref, sh_ref):
    xb = xb_ref[...]
    gu = _dot(xb, gu_ref[...])
    h = (_silu(gu[:, :FF]) * gu[:, FF:]).astype(jnp.bfloat16)
    sh = _dot(h, dw_ref[...])
    sgl = _dot(xb, sg_ref[...])
    sh_ref[...] = sh * jax.nn.sigmoid(sgl)


def _gemm_kernel(be_ref, xb_ref, po0_ref, po1_ref, va0_ref, va1_ref,
                 sh_ref, w1_ref, w2_ref, out_ref):
    del be_ref
    b = pl.program_id(0)
    si = b * BLK + jax.lax.broadcasted_iota(jnp.int32, (BLK, N), 0)
    eq0 = po0_ref[...] == si
    eq1 = po1_ref[...] == si
    pm = (eq0 | eq1).astype(jnp.bfloat16)
    xs = _dot(pm, xb_ref[...]).astype(jnp.bfloat16)
    h = _dot(xs, w1_ref[0])
    hb = _silu(h).astype(jnp.bfloat16)
    o = _dot(hb, w2_ref[0]).astype(jnp.bfloat16)
    q = (jnp.where(eq0, va0_ref[...], 0.0) +
         jnp.where(eq1, va1_ref[...], 0.0)).astype(jnp.bfloat16)
    contrib = jax.lax.dot_general(q, o, (((0,), (0,)), ((), ())),
                                  preferred_element_type=jnp.float32)

    @pl.when(b == 0)
    def _():
        out_ref[...] = sh_ref[...]

    out_ref[...] += contrib


def kernel(x, router_w, w1, w2, gate_up_w, down_w, shared_gate_w):
    Bv, Tv, Dv = x.shape
    flat = x.reshape(N, D)
    xb = flat.astype(jnp.bfloat16)

    po0, po1, va0, va1, blk_expert = pl.pallas_call(
        _router_kernel,
        grid=(1,),
        in_specs=[
            pl.BlockSpec((N, D), lambda t: (0, 0)),
            pl.BlockSpec((D, E), lambda t: (0, 0)),
        ],
        out_specs=[
            pl.BlockSpec((N, 1), lambda t: (0, 0)),
            pl.BlockSpec((N, 1), lambda t: (0, 0)),
            pl.BlockSpec((N, 1), lambda t: (0, 0)),
            pl.BlockSpec((N, 1), lambda t: (0, 0)),
            pl.BlockSpec((NBLK, 1), lambda t: (0, 0)),
        ],
        out_shape=[
            jax.ShapeDtypeStruct((N, 1), jnp.int32),
            jax.ShapeDtypeStruct((N, 1), jnp.int32),
            jax.ShapeDtypeStruct((N, 1), jnp.float32),
            jax.ShapeDtypeStruct((N, 1), jnp.float32),
            jax.ShapeDtypeStruct((NBLK, 1), jnp.int32),
        ],
    )(xb, router_w.astype(jnp.bfloat16))

    sh = pl.pallas_call(
        _shared_kernel,
        grid=(N // BT,),
        in_specs=[
            pl.BlockSpec((BT, D), lambda t: (t, 0)),
            pl.BlockSpec((D, 2 * FF), lambda t: (0, 0)),
            pl.BlockSpec((FF, D), lambda t: (0, 0)),
            pl.BlockSpec((D, 1), lambda t: (0, 0)),
        ],
        out_specs=pl.BlockSpec((BT, D), lambda t: (t, 0)),
        out_shape=jax.ShapeDtypeStruct((N, D), jnp.float32),
    )(xb, gate_up_w.astype(jnp.bfloat16), down_w.astype(jnp.bfloat16),
      shared_gate_w.astype(jnp.bfloat16))

    out = pl.pallas_call(
        _gemm_kernel,
        grid_spec=pltpu.PrefetchScalarGridSpec(
            num_scalar_prefetch=1,
            grid=(NBLK,),
            in_specs=[
                pl.BlockSpec((N, D), lambda b, be: (0, 0)),
                pl.BlockSpec((1, N), lambda b, be: (0, 0)),
                pl.BlockSpec((1, N), lambda b, be: (0, 0)),
                pl.BlockSpec((1, N), lambda b, be: (0, 0)),
                pl.BlockSpec((1, N), lambda b, be: (0, 0)),
                pl.BlockSpec((N, D), lambda b, be: (0, 0)),
                pl.BlockSpec((1, D, FF), lambda b, be: (be[b], 0, 0)),
                pl.BlockSpec((1, FF, D), lambda b, be: (be[b], 0, 0)),
            ],
            out_specs=pl.BlockSpec((N, D), lambda b, be: (0, 0)),
        ),
        out_shape=jax.ShapeDtypeStruct((N, D), jnp.float32),
        compiler_params=pltpu.CompilerParams(
            dimension_semantics=("arbitrary",)),
    )(blk_expert.reshape(NBLK), xb,
      po0.reshape(1, N), po1.reshape(1, N),
      va0.reshape(1, N), va1.reshape(1, N),
      sh, w1.astype(jnp.bfloat16), w2.astype(jnp.bfloat16))

    return out.reshape(Bv, Tv, Dv)


# speedup vs baseline: 3.2722x; 1.3606x over previous
import jax
import jax.numpy as jnp
from jax.experimental import pallas as pl
from jax.experimental.pallas import tpu as pltpu
from jax.experimental.pallas import tpu_sc as plsc

B, T, D = 1, 2048, 768
FF = 1536
E = 8
N = B * T
K = 2
NK = N * K

BLK = 128
NBLK = NK // BLK + E
PAD_N = NBLK * BLK
GW = 128
BT = 256


def _silu(v):
    return v * jax.nn.sigmoid(v)


def _dot(a, b):
    return jax.lax.dot_general(a, b, (((1,), (0,)), ((), ())),
                               preferred_element_type=jnp.float32)


def _sc_gather(table, indices):
    SPLIT = 2
    R, W0 = table.shape
    table = table.reshape(R * SPLIT, W0 // SPLIT)
    indices = (indices[:, None] * SPLIT +
               jnp.arange(SPLIT, dtype=jnp.int32)[None, :]).reshape(-1)
    M = indices.shape[0]
    W = table.shape[1]
    idx2 = indices.reshape(1, M)
    mesh = plsc.VectorSubcoreMesh(core_axis_name="core",
                                  subcore_axis_name="subcore")

    @pl.kernel(out_type=jax.ShapeDtypeStruct((M, W), table.dtype),
               mesh=mesh)
    def kern(x_hbm, i_hbm, o_hbm):
        def body(i_vmem, o_vmem):
            pltpu.sync_copy(x_hbm.at[i_vmem.at[0]], o_vmem)

        pltpu.emit_pipeline(
            body,
            grid=(M // GW,),
            in_specs=[pl.BlockSpec((1, GW), lambda i: (0, i))],
            out_specs=[pl.BlockSpec((GW, W), lambda i: (i, 0))],
            core_axis_name=("core", "subcore"),
            dimension_semantics=(pltpu.PARALLEL,),
        )(i_hbm, o_hbm)

    return kern(table, idx2).reshape(M // SPLIT, W0)


def _router_kernel(x_ref, rw_ref, po0_ref, po1_ref, va0_ref, va1_ref,
                   be_ref):
    xb = x_ref[...].astype(jnp.bfloat16)
    rwb = rw_ref[...].astype(jnp.bfloat16)
    logits = _dot(xb, rwb)
    m = jnp.max(logits, axis=-1, keepdims=True)
    p = jnp.exp(logits - m)
    p = p / jnp.sum(p, axis=-1, keepdims=True)
    iota = jax.lax.broadcasted_iota(jnp.int32, p.shape, 1)
    m1 = jnp.max(p, axis=-1, keepdims=True)
    i1 = jnp.min(jnp.where(p == m1, iota, E), axis=-1, keepdims=True)
    pm = jnp.where(iota == i1, -jnp.inf, p)
    m2 = jnp.max(pm, axis=-1, keepdims=True)
    i2 = jnp.min(jnp.where(pm == m2, iota, E), axis=-1, keepdims=True)
    va0_ref[...] = m1
    va1_ref[...] = m2
    oh1 = iota == i1
    oh2 = iota == i2
    ohb = (oh1 | oh2).astype(jnp.int32)
    incl = ohb
    d = 1
    while d < N:
        shifted = jnp.concatenate(
            [jnp.zeros((d, E), jnp.int32), incl[:N - d, :]], axis=0)
        incl = incl + shifted
        d *= 2
    excl = incl - ohb
    counts = incl[N - 1:N, :]
    pad_counts = ((counts + BLK - 1) // BLK) * BLK
    pad_end = pad_counts
    d = 1
    while d < E:
        pad_end = pad_end + jnp.concatenate(
            [jnp.zeros((1, d), jnp.int32), pad_end[:, :E - d]], axis=1)
        d *= 2
    pad_start = pad_end - pad_counts
    slot = excl + pad_start
    po0_ref[...] = jnp.sum(jnp.where(oh1, slot, 0), axis=1, keepdims=True)
    po1_ref[...] = jnp.sum(jnp.where(oh2, slot, 0), axis=1, keepdims=True)
    bb = BLK * jax.lax.broadcasted_iota(jnp.int32, (NBLK, E), 0)
    be = jnp.sum((jnp.broadcast_to(pad_end, (NBLK, E)) <= bb
                  ).astype(jnp.int32), axis=1, keepdims=True)
    be_ref[...] = jnp.minimum(be, E - 1)


def _shared_kernel(x_ref, gu_ref, dw_ref, sg_ref, sh_ref, gub_ref, dwb_ref):
    t = pl.program_id(0)

    @pl.when(t == 0)
    def _():
        gub_ref[...] = gu_ref[...].astype(jnp.bfloat16)
        dwb_ref[...] = dw_ref[...].astype(jnp.bfloat16)

    xb = x_ref[...].astype(jnp.bfloat16)
    gu = _dot(xb, gub_ref[...])
    h = (_silu(gu[:, :FF]) * gu[:, FF:]).astype(jnp.bfloat16)
    sh = _dot(h, dwb_ref[...])
    sgl = _dot(xb, sg_ref[...].astype(jnp.bfloat16))
    sh_ref[...] = sh * jax.nn.sigmoid(sgl)


def _gemm_kernel(be_ref, x_ref, po0_ref, po1_ref, va0_ref, va1_ref,
                 sh_ref, w1_ref, w2_ref, out_ref, xb_ref):
    del be_ref
    b = pl.program_id(0)

    @pl.when(b == 0)
    def _():
        xb_ref[...] = x_ref[...].astype(jnp.bfloat16)
        out_ref[...] = sh_ref[...]

    si = b * BLK + jax.lax.broadcasted_iota(jnp.int32, (BLK, N), 0)
    eq0 = po0_ref[...] == si
    eq1 = po1_ref[...] == si
    pm = (eq0 | eq1).astype(jnp.bfloat16)
    xs = _dot(pm, xb_ref[...]).astype(jnp.bfloat16)
    h = _dot(xs, w1_ref[0].astype(jnp.bfloat16))
    hb = _silu(h).astype(jnp.bfloat16)
    o = _dot(hb, w2_ref[0].astype(jnp.bfloat16)
             ).astype(jnp.bfloat16)
    q = (jnp.where(eq0, va0_ref[...], 0.0) +
         jnp.where(eq1, va1_ref[...], 0.0)).astype(jnp.bfloat16)
    contrib = jax.lax.dot_general(q, o, (((0,), (0,)), ((), ())),
                                  preferred_element_type=jnp.float32)
    out_ref[...] += contrib


def kernel(x, router_w, w1, w2, gate_up_w, down_w, shared_gate_w):
    Bv, Tv, Dv = x.shape
    flat = x.reshape(N, D)

    po0, po1, va0, va1, blk_expert = pl.pallas_call(
        _router_kernel,
        grid=(1,),
        in_specs=[
            pl.BlockSpec((N, D), lambda t: (0, 0)),
            pl.BlockSpec((D, E), lambda t: (0, 0)),
        ],
        out_specs=[
            pl.BlockSpec((N, 1), lambda t: (0, 0)),
            pl.BlockSpec((N, 1), lambda t: (0, 0)),
            pl.BlockSpec((N, 1), lambda t: (0, 0)),
            pl.BlockSpec((N, 1), lambda t: (0, 0)),
            pl.BlockSpec((NBLK, 1), lambda t: (0, 0)),
        ],
        out_shape=[
            jax.ShapeDtypeStruct((N, 1), jnp.int32),
            jax.ShapeDtypeStruct((N, 1), jnp.int32),
            jax.ShapeDtypeStruct((N, 1), jnp.float32),
            jax.ShapeDtypeStruct((N, 1), jnp.float32),
            jax.ShapeDtypeStruct((NBLK, 1), jnp.int32),
        ],
    )(flat, router_w)

    sh = pl.pallas_call(
        _shared_kernel,
        grid=(N // BT,),
        in_specs=[
            pl.BlockSpec((BT, D), lambda t: (t, 0)),
            pl.BlockSpec((D, 2 * FF), lambda t: (0, 0)),
            pl.BlockSpec((FF, D), lambda t: (0, 0)),
            pl.BlockSpec((D, 1), lambda t: (0, 0)),
        ],
        out_specs=pl.BlockSpec((BT, D), lambda t: (t, 0)),
        out_shape=jax.ShapeDtypeStruct((N, D), jnp.float32),
        scratch_shapes=[
            pltpu.VMEM((D, 2 * FF), jnp.bfloat16),
            pltpu.VMEM((FF, D), jnp.bfloat16),
        ],
    )(flat, gate_up_w, down_w, shared_gate_w)

    out = pl.pallas_call(
        _gemm_kernel,
        grid_spec=pltpu.PrefetchScalarGridSpec(
            num_scalar_prefetch=1,
            grid=(NBLK,),
            in_specs=[
                pl.BlockSpec((N, D), lambda b, be: (0, 0)),
                pl.BlockSpec((1, N), lambda b, be: (0, 0)),
                pl.BlockSpec((1, N), lambda b, be: (0, 0)),
                pl.BlockSpec((1, N), lambda b, be: (0, 0)),
                pl.BlockSpec((1, N), lambda b, be: (0, 0)),
                pl.BlockSpec((N, D), lambda b, be: (0, 0)),
                pl.BlockSpec((1, D, FF), lambda b, be: (be[b], 0, 0)),
                pl.BlockSpec((1, FF, D), lambda b, be: (be[b], 0, 0)),
            ],
            out_specs=pl.BlockSpec((N, D), lambda b, be: (0, 0)),
            scratch_shapes=[pltpu.VMEM((N, D), jnp.bfloat16)],
        ),
        out_shape=jax.ShapeDtypeStruct((N, D), jnp.float32),
        compiler_params=pltpu.CompilerParams(
            dimension_semantics=("arbitrary",)),
    )(blk_expert.reshape(NBLK), flat,
      po0.reshape(1, N), po1.reshape(1, N),
      va0.reshape(1, N), va1.reshape(1, N),
      sh, w1, w2)

    return out.reshape(Bv, Tv, Dv)


# speedup vs baseline: 4.0835x; 1.2479x over previous
import jax
import jax.numpy as jnp
from jax.experimental import pallas as pl
from jax.experimental.pallas import tpu as pltpu
from jax.experimental.pallas import tpu_sc as plsc

B, T, D = 1, 2048, 768
FF = 1536
E = 8
N = B * T
K = 2
NK = N * K

BLK = 256
NBLK = NK // BLK + E
PAD_N = NBLK * BLK
GW = 128
BT = 256


def _silu(v):
    return v * jax.nn.sigmoid(v)


def _dot(a, b):
    return jax.lax.dot_general(a, b, (((1,), (0,)), ((), ())),
                               preferred_element_type=jnp.float32)


def _sc_gather(table, indices):
    SPLIT = 2
    R, W0 = table.shape
    table = table.reshape(R * SPLIT, W0 // SPLIT)
    indices = (indices[:, None] * SPLIT +
               jnp.arange(SPLIT, dtype=jnp.int32)[None, :]).reshape(-1)
    M = indices.shape[0]
    W = table.shape[1]
    idx2 = indices.reshape(1, M)
    mesh = plsc.VectorSubcoreMesh(core_axis_name="core",
                                  subcore_axis_name="subcore")

    @pl.kernel(out_type=jax.ShapeDtypeStruct((M, W), table.dtype),
               mesh=mesh)
    def kern(x_hbm, i_hbm, o_hbm):
        def body(i_vmem, o_vmem):
            pltpu.sync_copy(x_hbm.at[i_vmem.at[0]], o_vmem)

        pltpu.emit_pipeline(
            body,
            grid=(M // GW,),
            in_specs=[pl.BlockSpec((1, GW), lambda i: (0, i))],
            out_specs=[pl.BlockSpec((GW, W), lambda i: (i, 0))],
            core_axis_name=("core", "subcore"),
            dimension_semantics=(pltpu.PARALLEL,),
        )(i_hbm, o_hbm)

    return kern(table, idx2).reshape(M // SPLIT, W0)


def _router_kernel(x_ref, rw_ref, po0_ref, po1_ref, va0_ref, va1_ref,
                   be_ref):
    xb = x_ref[...].astype(jnp.bfloat16)
    rwb = rw_ref[...].astype(jnp.bfloat16)
    logits = _dot(xb, rwb)
    m = jnp.max(logits, axis=-1, keepdims=True)
    p = jnp.exp(logits - m)
    p = p / jnp.sum(p, axis=-1, keepdims=True)
    iota = jax.lax.broadcasted_iota(jnp.int32, p.shape, 1)
    m1 = jnp.max(p, axis=-1, keepdims=True)
    i1 = jnp.min(jnp.where(p == m1, iota, E), axis=-1, keepdims=True)
    pm = jnp.where(iota == i1, -jnp.inf, p)
    m2 = jnp.max(pm, axis=-1, keepdims=True)
    i2 = jnp.min(jnp.where(pm == m2, iota, E), axis=-1, keepdims=True)
    va0_ref[...] = m1
    va1_ref[...] = m2
    oh1 = iota == i1
    oh2 = iota == i2
    ohb = (oh1 | oh2).astype(jnp.int32)
    incl = ohb
    d = 1
    while d < N:
        shifted = jnp.concatenate(
            [jnp.zeros((d, E), jnp.int32), incl[:N - d, :]], axis=0)
        incl = incl + shifted
        d *= 2
    excl = incl - ohb
    counts = incl[N - 1:N, :]
    pad_counts = ((counts + BLK - 1) // BLK) * BLK
    pad_end = pad_counts
    d = 1
    while d < E:
        pad_end = pad_end + jnp.concatenate(
            [jnp.zeros((1, d), jnp.int32), pad_end[:, :E - d]], axis=1)
        d *= 2
    pad_start = pad_end - pad_counts
    slot = excl + pad_start
    po0_ref[...] = jnp.sum(jnp.where(oh1, slot, 0), axis=1, keepdims=True)
    po1_ref[...] = jnp.sum(jnp.where(oh2, slot, 0), axis=1, keepdims=True)
    bb = BLK * jax.lax.broadcasted_iota(jnp.int32, (NBLK, E), 0)
    be = jnp.sum((jnp.broadcast_to(pad_end, (NBLK, E)) <= bb
                  ).astype(jnp.int32), axis=1, keepdims=True)
    nact = pad_end[:, E - 1:E] // BLK
    be_ref[...] = jnp.concatenate([jnp.minimum(be, E - 1), nact], axis=0)


def _shared_kernel(x_ref, gu_ref, dw_ref, sg_ref, sh_ref, gub_ref, dwb_ref):
    t = pl.program_id(0)

    @pl.when(t == 0)
    def _():
        gub_ref[...] = gu_ref[...].astype(jnp.bfloat16)
        dwb_ref[...] = dw_ref[...].astype(jnp.bfloat16)

    xb = x_ref[...].astype(jnp.bfloat16)
    gu = _dot(xb, gub_ref[...])
    h = (_silu(gu[:, :FF]) * gu[:, FF:]).astype(jnp.bfloat16)
    sh = _dot(h, dwb_ref[...])
    sgl = _dot(xb, sg_ref[...].astype(jnp.bfloat16))
    sh_ref[...] = sh * jax.nn.sigmoid(sgl)


def _gemm_kernel(be_ref, x_ref, po0_ref, po1_ref, va0_ref, va1_ref,
                 sh_ref, w1_ref, w2_ref, out_ref, xb_ref):
    b = pl.program_id(0)

    @pl.when(b == 0)
    def _():
        xb_ref[...] = x_ref[...].astype(jnp.bfloat16)
        out_ref[...] = sh_ref[...]

    @pl.when(b < be_ref[NBLK])
    def _():
        si = b * BLK + jax.lax.broadcasted_iota(jnp.int32, (BLK, N), 0)
        eq0 = po0_ref[...] == si
        eq1 = po1_ref[...] == si
        pm = (eq0 | eq1).astype(jnp.bfloat16)
        xs = _dot(pm, xb_ref[...]).astype(jnp.bfloat16)
        h = _dot(xs, w1_ref[0].astype(jnp.bfloat16))
        hb = _silu(h).astype(jnp.bfloat16)
        o = _dot(hb, w2_ref[0].astype(jnp.bfloat16)
                 ).astype(jnp.bfloat16)
        q = (jnp.where(eq0, va0_ref[...], 0.0) +
             jnp.where(eq1, va1_ref[...], 0.0)).astype(jnp.bfloat16)
        contrib = jax.lax.dot_general(q, o, (((0,), (0,)), ((), ())),
                                      preferred_element_type=jnp.float32)
        out_ref[...] += contrib


def kernel(x, router_w, w1, w2, gate_up_w, down_w, shared_gate_w):
    Bv, Tv, Dv = x.shape
    flat = x.reshape(N, D)

    po0, po1, va0, va1, blk_expert = pl.pallas_call(
        _router_kernel,
        grid=(1,),
        in_specs=[
            pl.BlockSpec((N, D), lambda t: (0, 0)),
            pl.BlockSpec((D, E), lambda t: (0, 0)),
        ],
        out_specs=[
            pl.BlockSpec((N, 1), lambda t: (0, 0)),
            pl.BlockSpec((N, 1), lambda t: (0, 0)),
            pl.BlockSpec((N, 1), lambda t: (0, 0)),
            pl.BlockSpec((N, 1), lambda t: (0, 0)),
            pl.BlockSpec((NBLK + 1, 1), lambda t: (0, 0)),
        ],
        out_shape=[
            jax.ShapeDtypeStruct((N, 1), jnp.int32),
            jax.ShapeDtypeStruct((N, 1), jnp.int32),
            jax.ShapeDtypeStruct((N, 1), jnp.float32),
            jax.ShapeDtypeStruct((N, 1), jnp.float32),
            jax.ShapeDtypeStruct((NBLK + 1, 1), jnp.int32),
        ],
    )(flat, router_w)

    sh = pl.pallas_call(
        _shared_kernel,
        grid=(N // BT,),
        in_specs=[
            pl.BlockSpec((BT, D), lambda t: (t, 0)),
            pl.BlockSpec((D, 2 * FF), lambda t: (0, 0)),
            pl.BlockSpec((FF, D), lambda t: (0, 0)),
            pl.BlockSpec((D, 1), lambda t: (0, 0)),
        ],
        out_specs=pl.BlockSpec((BT, D), lambda t: (t, 0)),
        out_shape=jax.ShapeDtypeStruct((N, D), jnp.float32),
        scratch_shapes=[
            pltpu.VMEM((D, 2 * FF), jnp.bfloat16),
            pltpu.VMEM((FF, D), jnp.bfloat16),
        ],
    )(flat, gate_up_w, down_w, shared_gate_w)

    out = pl.pallas_call(
        _gemm_kernel,
        grid_spec=pltpu.PrefetchScalarGridSpec(
            num_scalar_prefetch=1,
            grid=(NBLK,),
            in_specs=[
                pl.BlockSpec((N, D), lambda b, be: (0, 0)),
                pl.BlockSpec((1, N), lambda b, be: (0, 0)),
                pl.BlockSpec((1, N), lambda b, be: (0, 0)),
                pl.BlockSpec((1, N), lambda b, be: (0, 0)),
                pl.BlockSpec((1, N), lambda b, be: (0, 0)),
                pl.BlockSpec((N, D), lambda b, be: (0, 0)),
                pl.BlockSpec((1, D, FF), lambda b, be: (be[b], 0, 0)),
                pl.BlockSpec((1, FF, D), lambda b, be: (be[b], 0, 0)),
            ],
            out_specs=pl.BlockSpec((N, D), lambda b, be: (0, 0)),
            scratch_shapes=[pltpu.VMEM((N, D), jnp.bfloat16)],
        ),
        out_shape=jax.ShapeDtypeStruct((N, D), jnp.float32),
        compiler_params=pltpu.CompilerParams(
            dimension_semantics=("arbitrary",)),
    )(blk_expert.reshape(NBLK + 1), flat,
      po0.reshape(1, N), po1.reshape(1, N),
      va0.reshape(1, N), va1.reshape(1, N),
      sh, w1, w2)

    return out.reshape(Bv, Tv, Dv)
